# baseline (speedup 1.0000x reference)
import functools
import math

import jax
import jax.numpy as jnp
from jax import lax
from jax.experimental import pallas as pl
from jax.experimental.pallas import tpu as pltpu
from jax.experimental.pallas import tpu_sc as plsc

D_MODEL = 64
SCALE = math.sqrt(D_MODEL)
NC, NS, NW = 2, 16, 32
VOCAB = 1_000_000
NBLK = 7813
MAIN_ITERS = 244

_mesh = plsc.VectorSubcoreMesh(core_axis_name="c", subcore_axis_name="s")


def _worker_id():
    return lax.axis_index("s") * NC + lax.axis_index("c")


def _iota16():
    return lax.iota(jnp.int32, 16)


@functools.lru_cache(maxsize=None)
def _make_phase1():
    @functools.partial(
        pl.kernel,
        mesh=_mesh,
        compiler_params=pltpu.CompilerParams(needs_layout_passes=False),
        out_type=jax.ShapeDtypeStruct((VOCAB // 2, 128), jnp.float32),
        scratch_types=[
            pltpu.VMEM((8, 8, 128), jnp.float32),
            pltpu.VMEM((8, 8, 128), jnp.float32),
            pltpu.VMEM((64, 128), jnp.float32),
            pltpu.VMEM((64, 128), jnp.float32),
            pltpu.SemaphoreType.DMA,
            pltpu.SemaphoreType.DMA,
            pltpu.SemaphoreType.DMA,
            pltpu.SemaphoreType.DMA,
        ],
    )
    def k1(src, dst, ib0, ib1, ob0, ob1, si0, si1, so0, so1):
        w = _worker_id()
        ibs, obs, sis, sos = (ib0, ib1), (ob0, ob1), (si0, si1), (so0, so1)
        iota = _iota16()
        idx_c = [(iota + d0) // 8 for d0 in (0, 16, 32, 48)]
        idx_dl = [(iota + d0) % 8 for d0 in (0, 16, 32, 48)]
        zeros = jnp.zeros((16,), jnp.int32)

        def blk(i):
            return w + i * NW

        def start_in(i, b):
            pltpu.async_copy(
                src.at[:, :, pl.ds(blk(i) * 128, 128)], ibs[b], sis[b]
            )

        def transpose_block(ib, ob, n_rows):
            def row_body(r, carry):
                i_r = zeros + r
                p = r // 2
                qb = (r % 2) * 64
                for t, d0 in enumerate((0, 16, 32, 48)):
                    v = plsc.load_gather(ib, [idx_c[t], idx_dl[t], i_r])
                    ob[p, pl.ds(qb + d0, 16)] = v * SCALE
                return carry

            lax.fori_loop(0, n_rows, row_body, 0, unroll=2)

        start_in(0, 0)

        def body(i2, carry):
            for sub in range(2):
                i = i2 * 2 + sub
                pltpu.make_async_copy(
                    src.at[:, :, pl.ds(blk(i) * 128, 128)], ibs[sub], sis[sub]
                ).wait()

                @pl.when(i < MAIN_ITERS - 1)
                def _():
                    start_in(i + 1, 1 - sub)

                @pl.when(i2 > 0)
                def _():
                    pltpu.make_async_copy(
                        obs[sub], dst.at[pl.ds(blk(i) * 64, 64)], sos[sub]
                    ).wait()

                transpose_block(ibs[sub], obs[sub], 128)
                pltpu.async_copy(
                    obs[sub], dst.at[pl.ds(blk(i) * 64, 64)], sos[sub]
                )
            return carry

        lax.fori_loop(0, MAIN_ITERS // 2, body, 0)
        for sub in range(2):
            i = MAIN_ITERS - 2 + sub
            pltpu.make_async_copy(
                obs[sub], dst.at[pl.ds(blk(i) * 64, 64)], sos[sub]
            ).wait()

        @pl.when(w < NBLK - 7808)
        def _():
            j = 7808 + w
            pltpu.sync_copy(src.at[:, :, pl.ds(j * 128, 128)], ib0)

            @pl.when(w < 4)
            def _():
                transpose_block(ib0, ob0, 128)
                pltpu.sync_copy(ob0, dst.at[pl.ds(j * 64, 64)])

            @pl.when(w == 4)
            def _():
                transpose_block(ib0, ob0, 64)
                pltpu.sync_copy(
                    ob0.at[pl.ds(0, 32)], dst.at[pl.ds(j * 64, 32)]
                )

    return k1


@functools.lru_cache(maxsize=None)
def _make_phase2():
    @functools.partial(
        pl.kernel,
        mesh=_mesh,
        compiler_params=pltpu.CompilerParams(
            use_tc_tiling_on_sc=False, needs_layout_passes=False
        ),
        out_type=jax.ShapeDtypeStruct((200, 8, 32, 8, 128), jnp.float32),
        scratch_types=[
            pltpu.VMEM((200, 128), jnp.int32),
            pltpu.VMEM((128, 64), jnp.float32),
            pltpu.VMEM((128, 64), jnp.float32),
            pltpu.VMEM((8, 8, 128), jnp.float32),
            pltpu.VMEM((8, 8, 128), jnp.float32),
            pltpu.SemaphoreType.DMA,
            pltpu.SemaphoreType.DMA,
            pltpu.SemaphoreType.DMA,
            pltpu.SemaphoreType.DMA,
        ],
    )
    def k2(xb, lutr, o5, idxv, rb0, rb1, tb0, tb1, sg0, sg1, sw0, sw1):
        w = _worker_id()
        rbs, tbs, sgs, sws = (rb0, rb1), (tb0, tb1), (sg0, sg1), (sw0, sw1)
        iota = _iota16()
        i_rs = [iota + r0 for r0 in range(0, 128, 16)]
        zeros = jnp.zeros((16,), jnp.int32)

        pltpu.sync_copy(xb.at[w], idxv)

        def out_slice(i):
            g = w * 200 + i
            return o5.at[g // 32, :, g % 32]

        def start_gather(i, b):
            pltpu.async_copy(lutr.at[idxv.at[i]], rbs[b], sgs[b])

        def transpose_block(rb, tb):
            def c_body(c, carry):
                for dl in range(8):
                    i_d = zeros + (c * 8 + dl)
                    for t in range(8):
                        tb[c, dl, pl.ds(t * 16, 16)] = plsc.load_gather(
                            rb, [i_rs[t], i_d]
                        )
                return carry

            lax.fori_loop(0, 8, c_body, 0)

        start_gather(0, 0)

        def body(i2, carry):
            for sub in range(2):
                i = i2 * 2 + sub
                pltpu.make_async_copy(
                    lutr.at[idxv.at[i]], rbs[sub], sgs[sub]
                ).wait()

                @pl.when(i < 199)
                def _():
                    start_gather(i + 1, 1 - sub)

                @pl.when(i2 > 0)
                def _():
                    pltpu.make_async_copy(tbs[sub], out_slice(i), sws[sub]).wait()

                transpose_block(rbs[sub], tbs[sub])
                pltpu.async_copy(tbs[sub], out_slice(i), sws[sub])
            return carry

        lax.fori_loop(0, 100, body, 0)
        for sub in range(2):
            pltpu.make_async_copy(
                tbs[sub], out_slice(198 + sub), sws[sub]
            ).wait()

    return k2


def kernel(x, lut):
    lut_t3 = lut.T.reshape(8, 8, VOCAB)
    r128 = _make_phase1()(lut_t3)
    lutr = r128.reshape(VOCAB, D_MODEL)
    xb = x.astype(jnp.int32).T.reshape(NW, 200, 128)
    o5 = _make_phase2()(xb, lutr)
    return o5.transpose(2, 4, 0, 1, 3).reshape(4096, 200, D_MODEL)


# speedup vs baseline: 1.8982x; 1.8982x over previous
import functools
import math

import jax
import jax.numpy as jnp
from jax import lax
from jax.experimental import pallas as pl
from jax.experimental.pallas import tpu as pltpu
from jax.experimental.pallas import tpu_sc as plsc

D_MODEL = 64
SCALE = math.sqrt(D_MODEL)
NC, NS, NW = 2, 16, 32
VOCAB = 1_000_000
NBLK = 7813
MAIN_ITERS = 244

_mesh = plsc.VectorSubcoreMesh(core_axis_name="c", subcore_axis_name="s")


def _worker_id():
    return lax.axis_index("s") * NC + lax.axis_index("c")


@functools.lru_cache(maxsize=None)
def _make_phase1():
    @functools.partial(
        pl.kernel,
        mesh=_mesh,
        compiler_params=pltpu.CompilerParams(needs_layout_passes=False),
        out_type=jax.ShapeDtypeStruct((VOCAB * D_MODEL,), jnp.float32),
        scratch_types=[
            pltpu.VMEM((8, 8, 128), jnp.float32),
            pltpu.VMEM((8, 8, 128), jnp.float32),
            pltpu.VMEM((64 * 128,), jnp.float32),
            pltpu.VMEM((64 * 128,), jnp.float32),
            pltpu.SemaphoreType.DMA,
            pltpu.SemaphoreType.DMA,
            pltpu.SemaphoreType.DMA,
            pltpu.SemaphoreType.DMA,
        ],
    )
    def k1(src, dst, ib0, ib1, ob0, ob1, si0, si1, so0, so1):
        w = _worker_id()
        ibs, obs, sis, sos = (ib0, ib1), (ob0, ob1), (si0, si1), (so0, so1)
        iota = lax.iota(jnp.int32, 16)
        base = [
            ((iota + t * 16) // 2) * 128 + ((iota + t * 16) % 2) * 64
            for t in range(8)
        ]

        def blk(i):
            return w + i * NW

        def start_in(i, b):
            pltpu.async_copy(
                src.at[:, :, pl.ds(blk(i) * 128, 128)], ibs[b], sis[b]
            )

        def transpose_block(ib, ob, n_groups):
            @plsc.parallel_loop(0, 64, unroll=4)
            def _(d):
                c = d // 8
                dl = d % 8
                for t in range(n_groups):
                    v = ib[c, dl, pl.ds(t * 16, 16)] * SCALE
                    plsc.store_scatter(ob, [base[t] + d], v)

        start_in(0, 0)

        def body(i2, carry):
            for sub in range(2):
                i = i2 * 2 + sub
                pltpu.make_async_copy(
                    src.at[:, :, pl.ds(blk(i) * 128, 128)], ibs[sub], sis[sub]
                ).wait()

                @pl.when(i < MAIN_ITERS - 1)
                def _():
                    start_in(i + 1, 1 - sub)

                @pl.when(i2 > 0)
                def _():
                    pltpu.make_async_copy(
                        obs[sub], dst.at[pl.ds(blk(i) * 8192, 8192)], sos[sub]
                    ).wait()

                transpose_block(ibs[sub], obs[sub], 8)
                pltpu.async_copy(
                    obs[sub], dst.at[pl.ds(blk(i) * 8192, 8192)], sos[sub]
                )
            return carry

        lax.fori_loop(0, MAIN_ITERS // 2, body, 0)
        for sub in range(2):
            i = MAIN_ITERS - 2 + sub
            pltpu.make_async_copy(
                obs[sub], dst.at[pl.ds(blk(i) * 8192, 8192)], sos[sub]
            ).wait()

        @pl.when(w < NBLK - 7808)
        def _():
            j = 7808 + w
            pltpu.sync_copy(src.at[:, :, pl.ds(j * 128, 128)], ib0)

            @pl.when(w < 4)
            def _():
                transpose_block(ib0, ob0, 8)
                pltpu.sync_copy(ob0, dst.at[pl.ds(j * 8192, 8192)])

            @pl.when(w == 4)
            def _():
                transpose_block(ib0, ob0, 4)
                pltpu.sync_copy(
                    ob0.at[pl.ds(0, 4096)], dst.at[pl.ds(j * 8192, 4096)]
                )

    return k1


@functools.lru_cache(maxsize=None)
def _make_phase2():
    @functools.partial(
        pl.kernel,
        mesh=_mesh,
        compiler_params=pltpu.CompilerParams(
            use_tc_tiling_on_sc=False, needs_layout_passes=False
        ),
        out_type=jax.ShapeDtypeStruct((200, 8, 32, 8, 128), jnp.float32),
        scratch_types=[
            pltpu.VMEM((200, 128), jnp.int32),
            pltpu.VMEM((128, 64), jnp.float32),
            pltpu.VMEM((128, 64), jnp.float32),
            pltpu.VMEM((8, 8, 128), jnp.float32),
            pltpu.VMEM((8, 8, 128), jnp.float32),
            pltpu.SemaphoreType.DMA,
            pltpu.SemaphoreType.DMA,
            pltpu.SemaphoreType.DMA,
            pltpu.SemaphoreType.DMA,
        ],
    )
    def k2(xb, lutr, o5, idxv, rb0, rb1, tb0, tb1, sg0, sg1, sw0, sw1):
        w = _worker_id()
        rbs, tbs, sgs, sws = (rb0, rb1), (tb0, tb1), (sg0, sg1), (sw0, sw1)
        iota = lax.iota(jnp.int32, 16)
        i_rs = [iota + t * 16 for t in range(8)]
        zeros = jnp.zeros((16,), jnp.int32)

        pltpu.sync_copy(xb.at[w], idxv)

        def out_slice(i):
            g = w * 200 + i
            return o5.at[g // 32, :, g % 32]

        def start_gather(i, b):
            pltpu.async_copy(lutr.at[idxv.at[i]], rbs[b], sgs[b])

        def transpose_block(rb, tb):
            @plsc.parallel_loop(0, 64, unroll=4)
            def _(d):
                c = d // 8
                dl = d % 8
                i_d = zeros + d
                for t in range(8):
                    tb[c, dl, pl.ds(t * 16, 16)] = plsc.load_gather(
                        rb, [i_rs[t], i_d]
                    )

        start_gather(0, 0)

        def body(i2, carry):
            for sub in range(2):
                i = i2 * 2 + sub
                pltpu.make_async_copy(
                    lutr.at[idxv.at[i]], rbs[sub], sgs[sub]
                ).wait()

                @pl.when(i < 199)
                def _():
                    start_gather(i + 1, 1 - sub)

                @pl.when(i2 > 0)
                def _():
                    pltpu.make_async_copy(tbs[sub], out_slice(i), sws[sub]).wait()

                transpose_block(rbs[sub], tbs[sub])
                pltpu.async_copy(tbs[sub], out_slice(i), sws[sub])
            return carry

        lax.fori_loop(0, 100, body, 0)
        for sub in range(2):
            pltpu.make_async_copy(
                tbs[sub], out_slice(198 + sub), sws[sub]
            ).wait()

    return k2


def kernel(x, lut):
    lut_t3 = lut.T.reshape(8, 8, VOCAB)
    flat = _make_phase1()(lut_t3)
    lutr = flat.reshape(VOCAB, D_MODEL)
    xb = x.astype(jnp.int32).T.reshape(NW, 200, 128)
    o5 = _make_phase2()(xb, lutr)
    return o5.transpose(2, 4, 0, 1, 3).reshape(4096, 200, D_MODEL)


# speedup vs baseline: 2.7647x; 1.4565x over previous
import functools
import math

import jax
import jax.numpy as jnp
from jax import lax
from jax.experimental import pallas as pl
from jax.experimental.pallas import tpu as pltpu
from jax.experimental.pallas import tpu_sc as plsc

D_MODEL = 64
SCALE = math.sqrt(D_MODEL)
NC, NS, NW = 2, 16, 32
VOCAB = 1_000_000
NBLK = 7813
MAIN_ITERS = 244
PAD = 129

_mesh = plsc.VectorSubcoreMesh(core_axis_name="c", subcore_axis_name="s")


def _worker_id():
    return lax.axis_index("s") * NC + lax.axis_index("c")


def _didx():
    iota = lax.iota(jnp.int32, 16)
    idx_c = [(iota + d0) // 8 for d0 in (0, 16, 32, 48)]
    idx_dl = [(iota + d0) % 8 for d0 in (0, 16, 32, 48)]
    return idx_c, idx_dl


@functools.lru_cache(maxsize=None)
def _make_phase1():
    @functools.partial(
        pl.kernel,
        mesh=_mesh,
        compiler_params=pltpu.CompilerParams(needs_layout_passes=False),
        out_type=jax.ShapeDtypeStruct((VOCAB // 2, 128), jnp.float32),
        scratch_types=[
            pltpu.VMEM((8, 8, PAD), jnp.float32),
            pltpu.VMEM((8, 8, PAD), jnp.float32),
            pltpu.VMEM((64, 128), jnp.float32),
            pltpu.VMEM((64, 128), jnp.float32),
            pltpu.SemaphoreType.DMA,
            pltpu.SemaphoreType.DMA,
            pltpu.SemaphoreType.DMA,
            pltpu.SemaphoreType.DMA,
        ],
    )
    def k1(src, dst, ib0, ib1, ob0, ob1, si0, si1, so0, so1):
        w = _worker_id()
        ibs, obs, sis, sos = (ib0, ib1), (ob0, ob1), (si0, si1), (so0, so1)
        idx_c, idx_dl = _didx()
        zeros = jnp.zeros((16,), jnp.int32)

        def blk(i):
            return w + i * NW

        def start_in(i, b):
            pltpu.async_copy(
                src.at[:, :, pl.ds(blk(i) * 128, 128)],
                ibs[b].at[:, :, pl.ds(0, 128)],
                sis[b],
            )

        def transpose_block(ib, ob, n_rows):
            @plsc.parallel_loop(0, n_rows, unroll=4)
            def _(r):
                i_r = zeros + r
                p = r // 2
                qb = (r % 2) * 64
                for t, d0 in enumerate((0, 16, 32, 48)):
                    v = plsc.load_gather(ib, [idx_c[t], idx_dl[t], i_r])
                    ob[p, pl.ds(qb + d0, 16)] = v * SCALE

        start_in(0, 0)

        def body(i2, carry):
            for sub in range(2):
                i = i2 * 2 + sub
                pltpu.make_async_copy(
                    src.at[:, :, pl.ds(blk(i) * 128, 128)],
                    ibs[sub].at[:, :, pl.ds(0, 128)],
                    sis[sub],
                ).wait()

                @pl.when(i < MAIN_ITERS - 1)
                def _():
                    start_in(i + 1, 1 - sub)

                @pl.when(i2 > 0)
                def _():
                    pltpu.make_async_copy(
                        obs[sub], dst.at[pl.ds(blk(i) * 64, 64)], sos[sub]
                    ).wait()

                transpose_block(ibs[sub], obs[sub], 128)
                pltpu.async_copy(
                    obs[sub], dst.at[pl.ds(blk(i) * 64, 64)], sos[sub]
                )
            return carry

        lax.fori_loop(0, MAIN_ITERS // 2, body, 0)
        for sub in range(2):
            i = MAIN_ITERS - 2 + sub
            pltpu.make_async_copy(
                obs[sub], dst.at[pl.ds(blk(i) * 64, 64)], sos[sub]
            ).wait()

        @pl.when(w < NBLK - 7808)
        def _():
            j = 7808 + w
            pltpu.sync_copy(
                src.at[:, :, pl.ds(j * 128, 128)], ib0.at[:, :, pl.ds(0, 128)]
            )

            @pl.when(w < 4)
            def _():
                transpose_block(ib0, ob0, 128)
                pltpu.sync_copy(ob0, dst.at[pl.ds(j * 64, 64)])

            @pl.when(w == 4)
            def _():
                transpose_block(ib0, ob0, 64)
                pltpu.sync_copy(
                    ob0.at[pl.ds(0, 32)], dst.at[pl.ds(j * 64, 32)]
                )

    return k1


@functools.lru_cache(maxsize=None)
def _make_phase2():
    @functools.partial(
        pl.kernel,
        mesh=_mesh,
        compiler_params=pltpu.CompilerParams(
            use_tc_tiling_on_sc=False, needs_layout_passes=False
        ),
        out_type=jax.ShapeDtypeStruct((200, 8, 32, 8, 128), jnp.float32),
        scratch_types=[
            pltpu.VMEM((200, 128), jnp.int32),
            pltpu.VMEM((128, 64), jnp.float32),
            pltpu.VMEM((128, 64), jnp.float32),
            pltpu.VMEM((8, 8, PAD), jnp.float32),
            pltpu.VMEM((8, 8, PAD), jnp.float32),
            pltpu.SemaphoreType.DMA,
            pltpu.SemaphoreType.DMA,
            pltpu.SemaphoreType.DMA,
            pltpu.SemaphoreType.DMA,
        ],
    )
    def k2(xb, lutr, o5, idxv, rb0, rb1, tb0, tb1, sg0, sg1, sw0, sw1):
        w = _worker_id()
        rbs, tbs, sgs, sws = (rb0, rb1), (tb0, tb1), (sg0, sg1), (sw0, sw1)
        idx_c, idx_dl = _didx()
        zeros = jnp.zeros((16,), jnp.int32)

        pltpu.sync_copy(xb.at[w], idxv)

        def out_slice(i):
            g = w * 200 + i
            return o5.at[g // 32, :, g % 32]

        def start_gather(i, b):
            pltpu.async_copy(lutr.at[idxv.at[i]], rbs[b], sgs[b])

        def transpose_block(rb, tb):
            @plsc.parallel_loop(0, 128, unroll=4)
            def _(r):
                i_r = zeros + r
                for t, d0 in enumerate((0, 16, 32, 48)):
                    v = rb[r, pl.ds(d0, 16)]
                    plsc.store_scatter(tb, [idx_c[t], idx_dl[t], i_r], v)

        start_gather(0, 0)

        def body(i2, carry):
            for sub in range(2):
                i = i2 * 2 + sub
                pltpu.make_async_copy(
                    lutr.at[idxv.at[i]], rbs[sub], sgs[sub]
                ).wait()

                @pl.when(i < 199)
                def _():
                    start_gather(i + 1, 1 - sub)

                @pl.when(i2 > 0)
                def _():
                    pltpu.make_async_copy(
                        tbs[sub].at[:, :, pl.ds(0, 128)], out_slice(i), sws[sub]
                    ).wait()

                transpose_block(rbs[sub], tbs[sub])
                pltpu.async_copy(
                    tbs[sub].at[:, :, pl.ds(0, 128)], out_slice(i), sws[sub]
                )
            return carry

        lax.fori_loop(0, 100, body, 0)
        for sub in range(2):
            pltpu.make_async_copy(
                tbs[sub].at[:, :, pl.ds(0, 128)], out_slice(198 + sub), sws[sub]
            ).wait()

    return k2


def kernel(x, lut):
    lut_t3 = lut.T.reshape(8, 8, VOCAB)
    r128 = _make_phase1()(lut_t3)
    lutr = r128.reshape(VOCAB, D_MODEL)
    xb = x.astype(jnp.int32).T.reshape(NW, 200, 128)
    o5 = _make_phase2()(xb, lutr)
    return o5.transpose(2, 4, 0, 1, 3).reshape(4096, 200, D_MODEL)
